# baseline (speedup 1.0000x reference)
import functools

import jax
import jax.numpy as jnp
from jax import lax
from jax.experimental import pallas as pl
from jax.experimental.pallas import tpu as pltpu

_N_BINS = 15
_LOG2E = 1.4426950408889634
_BLK = 4000


def _partials_kernel(logits_ref, lab_ref, part_ref):
    x = logits_ref[...]
    lab_row = lab_ref[0]
    b, c = x.shape

    lab = jnp.transpose(lab_row)
    idx = lax.broadcasted_iota(jnp.int32, (b, c), 1)
    onehot = (idx == lab)
    m = jnp.max(x, axis=1, keepdims=True)
    s = jnp.sum(jnp.exp2((x - m) * _LOG2E), axis=1, keepdims=True)
    g = jnp.sum(jnp.where(onehot, x, 0.0), axis=1, keepdims=True)
    conf = 1.0 / s
    acc = (g == m).astype(jnp.float32)

    ii = lax.broadcasted_iota(jnp.int32, (1, _N_BINS), 1).astype(jnp.float32)
    lo = ii / _N_BINS
    hi = (ii + 1.0) / _N_BINS
    mask = ((conf > lo) & (conf <= hi)).astype(jnp.float32)
    part_ref[0, 0, :] = jnp.sum(mask, axis=0)
    part_ref[0, 1, :] = jnp.sum(conf * mask, axis=0)
    part_ref[0, 2, :] = jnp.sum(acc * mask, axis=0)


def _fold_kernel(part_ref, out_ref, *, n_total):
    p = part_ref[...]
    cnt = jnp.sum(p[:, 0, :], axis=0)
    sconf = jnp.sum(p[:, 1, :], axis=0)
    sacc = jnp.sum(p[:, 2, :], axis=0)
    safe = jnp.maximum(cnt, 1.0)
    avg_conf = sconf / safe
    avg_acc = sacc / safe
    prop = cnt / n_total
    contrib = jnp.abs(avg_conf - avg_acc) * prop
    out_ref[...] = jnp.sum(jnp.where(prop > 0, contrib, 0.0)).reshape(1, 1)


def kernel(logits, labels):
    n, c = logits.shape
    blk = _BLK
    n_blocks = n // blk
    labels2 = labels.astype(jnp.int32).reshape(n_blocks, 1, blk)
    partials = pl.pallas_call(
        _partials_kernel,
        grid=(n_blocks,),
        in_specs=[
            pl.BlockSpec((blk, c), lambda i: (i, 0)),
            pl.BlockSpec((1, 1, blk), lambda i: (i, 0, 0)),
        ],
        out_specs=pl.BlockSpec((1, 3, _N_BINS), lambda i: (i, 0, 0)),
        out_shape=jax.ShapeDtypeStruct((n_blocks, 3, _N_BINS), jnp.float32),
        compiler_params=pltpu.CompilerParams(
            dimension_semantics=("parallel",)),
    )(logits, labels2)
    out = pl.pallas_call(
        functools.partial(_fold_kernel, n_total=float(n)),
        out_shape=jax.ShapeDtypeStruct((1, 1), jnp.float32),
    )(partials)
    return out.reshape(1)
